# baseline (speedup 1.0000x reference)
import jax
import jax.numpy as jnp
from jax import lax
from jax.experimental import pallas as pl
from jax.experimental.pallas import tpu as pltpu
from jax.experimental.pallas import tpu_sc as plsc

N = 5000
G = 16
NP = 5120
PER_TILE = 320
CHUNKS = PER_TILE // 16
TAIL = N - 15 * PER_TILE

C1 = 1.3132616875182228
C2 = 0.6931471805599453
LN2 = 0.6931471805599453
SQRT2 = 1.4142135623730951

_f32 = jnp.float32
_i32 = jnp.int32

_INB = "promise_in_bounds"


def _b(s):
    return jnp.broadcast_to(s, (16,))


def _take(v, idx):
    return v.at[idx].get(mode=_INB)


def _ln(x):
    bits = lax.bitcast_convert_type(x, _i32)
    e = (bits >> 23) & 0xFF
    m = lax.bitcast_convert_type((bits & 0x007FFFFF) | 0x3F800000, _f32)
    big = m >= _f32(SQRT2)
    m = jnp.where(big, m * _f32(0.5), m)
    ef = (e - 127).astype(_f32) + jnp.where(big, _f32(1.0), _f32(0.0))
    t = (m - _f32(1.0)) / (m + _f32(1.0))
    t2 = t * t
    p = t * (_f32(2.0) + t2 * (_f32(2.0 / 3.0) + t2 * (_f32(2.0 / 5.0)
             + t2 * (_f32(2.0 / 7.0) + t2 * _f32(2.0 / 9.0)))))
    return p + ef * _f32(LN2)


def _sl1(d):
    a = jnp.abs(d)
    m = jnp.minimum(a, _f32(1.0))
    return _f32(0.5) * m * (a + a - m)


def _body(rflath, aflath, sch, gflath,
          loss_out, idx_out,
          rblk, ablk,
          rxv, ryv, rx2v, ry2v, axv, ayv, ax2v, ay2v, scv,
          ioub, rmaxb, partv, shared, allv,
          lossov, idxov, sem):
    cid = lax.axis_index("c")
    sid = lax.axis_index("s")

    @pl.when(cid == 0)
    def _core0():
        li = lax.iota(_i32, 16)
        lif = li.astype(_f32)

        base = sid * PER_TILE
        basef = base.astype(_f32)

        gcp = pltpu.async_copy(gflath, partv.at[pl.ds(0, 64)], sem)

        @pl.when(sid < 15)
        def _full_dma():
            cps = [
                pltpu.async_copy(rflath.at[pl.ds(base * 4, PER_TILE * 4)],
                                 rblk, sem),
                pltpu.async_copy(aflath.at[pl.ds(base * 4, PER_TILE * 4)],
                                 ablk, sem),
                pltpu.async_copy(sch.at[pl.ds(base, PER_TILE)], scv, sem),
            ]
            for c in cps:
                c.wait()

        @pl.when(sid == 15)
        def _tail_dma():
            cps = [
                pltpu.async_copy(rflath.at[pl.ds(base * 4, TAIL * 4)],
                                 rblk.at[pl.ds(0, TAIL * 4)], sem),
                pltpu.async_copy(aflath.at[pl.ds(base * 4, TAIL * 4)],
                                 ablk.at[pl.ds(0, TAIL * 4)], sem),
                pltpu.async_copy(sch.at[pl.ds(base, TAIL)],
                                 scv.at[pl.ds(0, TAIL)], sem),
            ]
            for c in cps:
                c.wait()

        gcp.wait()
        i4 = li * 4
        gx1 = plsc.load_gather(partv, [i4])
        gy1 = plsc.load_gather(partv, [i4 + 1])
        gx2 = plsc.load_gather(partv, [i4 + 2])
        gy2 = plsc.load_gather(partv, [i4 + 3])

        for c in range(CHUNKS):
            s16 = c * 16
            j4 = i4 + c * 64
            axv[pl.ds(s16, 16)] = plsc.load_gather(ablk, [j4])
            ayv[pl.ds(s16, 16)] = plsc.load_gather(ablk, [j4 + 1])
            ax2v[pl.ds(s16, 16)] = plsc.load_gather(ablk, [j4 + 2])
            ay2v[pl.ds(s16, 16)] = plsc.load_gather(ablk, [j4 + 3])
            rxv[pl.ds(s16, 16)] = plsc.load_gather(rblk, [j4])
            ryv[pl.ds(s16, 16)] = plsc.load_gather(rblk, [j4 + 1])
            rx2v[pl.ds(s16, 16)] = plsc.load_gather(rblk, [j4 + 2])
            ry2v[pl.ds(s16, 16)] = plsc.load_gather(rblk, [j4 + 3])

        gw = gx2 - gx1
        gh = gy2 - gy1
        gav = gw * gh
        gcx = (gx1 + gx2) * _f32(0.5)
        gcy = (gy1 + gy2) * _f32(0.5)
        lgw = _ln(gw)
        lgh = _ln(gh)

        ones = _b(_f32(1.0))
        zeros = _b(_f32(0.0))
        cgs = [jnp.full((16,), g, _i32) for g in range(G)]

        def slow(cnt_a, regsum_a, cx1, cy1, cx2, cy2,
                 bx1, by1, bx2, by2, valid, start):
            rw = cx2 - cx1
            rh = cy2 - cy1
            rcx = (cx1 + cx2) * _f32(0.5)
            rcy = (cy1 + cy2) * _f32(0.5)
            lrw = _ln(jnp.where(valid, rw, ones))
            lrh = _ln(jnp.where(valid, rh, ones))
            invaw = ones / jnp.where(valid, bx2 - bx1, ones)
            invah = ones / jnp.where(valid, by2 - by1, ones)
            for g in range(G):
                iou = ioub[pl.ds(g * PER_TILE + start, 16)]
                pos = iou > _f32(0.8)
                cnt_a = cnt_a + jnp.where(pos, ones, zeros)
                dx = (rcx - _take(gcx, cgs[g])) * invaw
                dy = (rcy - _take(gcy, cgs[g])) * invah
                dw = lrw - _take(lgw, cgs[g])
                dh = lrh - _take(lgh, cgs[g])
                pp = (_sl1(dx) + _sl1(dy) + _sl1(dw) + _sl1(dh)) * _f32(0.25)
                regsum_a = regsum_a + jnp.where(pos, pp, zeros)
            return cnt_a, regsum_a

        def fast(cnt_a, regsum_a, cx1, cy1, cx2, cy2,
                 bx1, by1, bx2, by2, valid, start):
            return cnt_a, regsum_a

        def chunk(c, carry):
            (cnt_a, regsum_a, npos_a, nneg_a, clspos_a) = carry
            start = pl.multiple_of(c * 16, 16)
            rowf = lif + basef + start.astype(_f32)
            valid = rowf < _f32(N)

            cx1 = rxv[pl.ds(start, 16)]
            cy1 = ryv[pl.ds(start, 16)]
            cx2 = rx2v[pl.ds(start, 16)]
            cy2 = ry2v[pl.ds(start, 16)]
            bx1 = axv[pl.ds(start, 16)]
            by1 = ayv[pl.ds(start, 16)]
            bx2 = ax2v[pl.ds(start, 16)]
            by2 = ay2v[pl.ds(start, 16)]
            scc = scv[pl.ds(start, 16)]
            arear = (cx2 - cx1) * (cy2 - cy1)

            rmax = -ones
            for g in range(G):
                iw = (jnp.minimum(cx2, _take(gx2, cgs[g]))
                      - jnp.maximum(cx1, _take(gx1, cgs[g])))
                ih = (jnp.minimum(cy2, _take(gy2, cgs[g]))
                      - jnp.maximum(cy1, _take(gy1, cgs[g])))
                inter = jnp.maximum(iw, zeros) * jnp.maximum(ih, zeros)
                iou = inter / (arear + _take(gav, cgs[g]) - inter)
                iou = jnp.where(valid, iou, zeros)
                ioub[pl.ds(g * PER_TILE + start, 16)] = iou
                rmax = jnp.maximum(rmax, iou)
            rmaxb[pl.ds(start, 16)] = rmax

            pv = (rmax > _f32(0.8)) & valid
            nv = (rmax < _f32(0.3)) & valid
            npos_a = npos_a + jnp.where(pv, ones, zeros)
            clspos_a = clspos_a + jnp.where(pv, _f32(C1) - scc, zeros)
            nneg_a = nneg_a + jnp.where(nv, ones, zeros)

            cnt_a, regsum_a = lax.cond(
                jnp.any(rmax > _f32(0.8)), slow, fast,
                cnt_a, regsum_a, cx1, cy1, cx2, cy2,
                bx1, by1, bx2, by2, valid, start)
            return (cnt_a, regsum_a, npos_a, nneg_a, clspos_a)

        init = (zeros, zeros, zeros, zeros, zeros)
        (cnt_a, regsum_a, npos_a, nneg_a, clspos_a) = \
            lax.fori_loop(0, CHUNKS, chunk, init)

        colmax = -ones
        colloc = zeros
        big = _b(_f32(1e9))
        for g in range(G):
            maxv = ioub[pl.ds(g * PER_TILE, 16)]
            idxv = lif
            for c in range(1, CHUNKS):
                v = ioub[pl.ds(g * PER_TILE + c * 16, 16)]
                u = v > maxv
                maxv = jnp.where(u, v, maxv)
                idxv = jnp.where(u, lif + _f32(c * 16), idxv)
            m = jnp.max(maxv)
            bm = _b(m)
            fi = jnp.min(jnp.where(maxv >= bm, idxv, big))
            lm = li == g
            colmax = jnp.where(lm, bm, colmax)
            colloc = jnp.where(lm, _b(fi), colloc)

        lrowi = colloc.astype(_i32)
        colidx = colloc + _b(basef)
        colrm = plsc.load_gather(rmaxb, [lrowi])
        colsc = plsc.load_gather(scv, [lrowi])

        px1 = plsc.load_gather(rxv, [lrowi])
        py1 = plsc.load_gather(ryv, [lrowi])
        px2 = plsc.load_gather(rx2v, [lrowi])
        py2 = plsc.load_gather(ry2v, [lrowi])
        qx1 = plsc.load_gather(axv, [lrowi])
        qy1 = plsc.load_gather(ayv, [lrowi])
        qx2 = plsc.load_gather(ax2v, [lrowi])
        qy2 = plsc.load_gather(ay2v, [lrowi])
        aw = qx2 - qx1
        ah = qy2 - qy1
        dx = ((px1 + px2) * _f32(0.5) - gcx) / aw
        dy = ((py1 + py2) * _f32(0.5) - gcy) / ah
        dw = _ln(px2 - px1) - lgw
        dh = _ln(py2 - py1) - lgh
        colpp = (_sl1(dx) + _sl1(dy) + _sl1(dw) + _sl1(dh)) * _f32(0.25)

        partv[pl.ds(0, 16)] = colmax
        partv[pl.ds(16, 16)] = colidx
        partv[pl.ds(32, 16)] = colpp
        partv[pl.ds(48, 16)] = colsc
        partv[pl.ds(64, 16)] = colrm
        partv[pl.ds(80, 16)] = cnt_a
        partv[pl.ds(96, 16)] = regsum_a
        partv[pl.ds(112, 16)] = npos_a
        partv[pl.ds(128, 16)] = nneg_a
        partv[pl.ds(144, 16)] = clspos_a
        pltpu.sync_copy(partv, shared.at[pl.ds(sid * 160, 160)])
        plsc.subcore_barrier()

        @pl.when(sid == 0)
        def _final():
            pltpu.sync_copy(shared, allv)
            mcolmax = allv[pl.ds(0, 16)]
            mcolidx = allv[pl.ds(16, 16)]
            mcolpp = allv[pl.ds(32, 16)]
            mcolsc = allv[pl.ds(48, 16)]
            mcolrm = allv[pl.ds(64, 16)]
            mcnt = allv[pl.ds(80, 16)]
            mregsum = allv[pl.ds(96, 16)]
            mnpos = allv[pl.ds(112, 16)]
            mnneg = allv[pl.ds(128, 16)]
            mclspos = allv[pl.ds(144, 16)]
            for t in range(1, 16):
                tm = allv[pl.ds(t * 160 + 0, 16)]
                u = tm > mcolmax
                mcolmax = jnp.where(u, tm, mcolmax)
                mcolidx = jnp.where(u, allv[pl.ds(t * 160 + 16, 16)], mcolidx)
                mcolpp = jnp.where(u, allv[pl.ds(t * 160 + 32, 16)], mcolpp)
                mcolsc = jnp.where(u, allv[pl.ds(t * 160 + 48, 16)], mcolsc)
                mcolrm = jnp.where(u, allv[pl.ds(t * 160 + 64, 16)], mcolrm)
                mcnt = mcnt + allv[pl.ds(t * 160 + 80, 16)]
                mregsum = mregsum + allv[pl.ds(t * 160 + 96, 16)]
                mnpos = mnpos + allv[pl.ds(t * 160 + 112, 16)]
                mnneg = mnneg + allv[pl.ds(t * 160 + 128, 16)]
                mclspos = mclspos + allv[pl.ds(t * 160 + 144, 16)]

            best = mcolidx.astype(_i32)

            notpos = mcolmax <= _f32(0.8)
            d_cntr = jnp.sum(jnp.where(notpos, ones, zeros))
            d_regsum = jnp.sum(jnp.where(notpos, mcolpp, zeros))

            dup = li < 0
            for k in range(G - 1):
                bk = _b(best[k])
                dup = dup | ((best == bk) & (li > k))
            firstm = ~dup

            anyposb = mcolrm > _f32(0.8)
            allnegb = mcolrm < _f32(0.3)
            notany = firstm & (~anyposb)
            d_npos = jnp.sum(jnp.where(notany, ones, zeros))
            d_clspos = jnp.sum(jnp.where(notany, _f32(C1) - mcolsc, zeros))
            d_nneg = jnp.sum(jnp.where(firstm & allnegb, ones, zeros))

            npos_v = _b(jnp.sum(mnpos)) + _b(d_npos)
            nneg_v = _b(jnp.sum(mnneg)) - _b(d_nneg)
            nsel = npos_v + nneg_v
            cls = _b(jnp.sum(mclspos)) + _b(d_clspos) + nneg_v * _f32(C2)
            cntr = _b(jnp.sum(mcnt)) + _b(d_cntr)
            regs = _b(jnp.sum(mregsum)) + _b(d_regsum)
            lossv = cls / nsel / nsel + _f32(10.0) * regs / cntr

            lossov[...] = lossv
            idxov[...] = _b(best[G - 1])
            pltpu.sync_copy(lossov, loss_out)
            pltpu.sync_copy(idxov, idx_out)


@jax.jit
def kernel(scores, regions, anchors, ground_truth_boxes):
    rflat = regions.reshape(-1)
    aflat = anchors.reshape(-1)
    gflat = ground_truth_boxes.reshape(-1)

    mesh = plsc.VectorSubcoreMesh(core_axis_name="c", subcore_axis_name="s",
                                  num_cores=1)
    f = pl.kernel(
        _body,
        out_type=(
            jax.ShapeDtypeStruct((16,), _f32),
            jax.ShapeDtypeStruct((16,), _i32),
        ),
        mesh=mesh,
        compiler_params=pltpu.CompilerParams(needs_layout_passes=False,
                                             skip_device_barrier=True),
        scratch_types=[
            pltpu.VMEM((PER_TILE * 4,), _f32),
            pltpu.VMEM((PER_TILE * 4,), _f32),
            pltpu.VMEM((PER_TILE,), _f32),
            pltpu.VMEM((PER_TILE,), _f32),
            pltpu.VMEM((PER_TILE,), _f32),
            pltpu.VMEM((PER_TILE,), _f32),
            pltpu.VMEM((PER_TILE,), _f32),
            pltpu.VMEM((PER_TILE,), _f32),
            pltpu.VMEM((PER_TILE,), _f32),
            pltpu.VMEM((PER_TILE,), _f32),
            pltpu.VMEM((PER_TILE,), _f32),
            pltpu.VMEM((G * PER_TILE,), _f32),
            pltpu.VMEM((PER_TILE,), _f32),
            pltpu.VMEM((160,), _f32),
            pltpu.VMEM_SHARED((2560,), _f32),
            pltpu.VMEM((2560,), _f32),
            pltpu.VMEM((16,), _f32),
            pltpu.VMEM((16,), _i32),
            pltpu.SemaphoreType.DMA,
        ],
    )
    loss_v, idx_v = f(rflat, aflat, scores, gflat)
    return loss_v[0], idx_v[0]


# speedup vs baseline: 1.1522x; 1.1522x over previous
import jax
import jax.numpy as jnp
from jax import lax
from jax.experimental import pallas as pl
from jax.experimental.pallas import tpu as pltpu
from jax.experimental.pallas import tpu_sc as plsc

N = 5000
G = 16
NP = 5120
PER_TILE = 320
CHUNKS = PER_TILE // 16

C1 = 1.3132616875182228
C2 = 0.6931471805599453
LN2 = 0.6931471805599453
SQRT2 = 1.4142135623730951

_f32 = jnp.float32
_i32 = jnp.int32

_INB = "promise_in_bounds"


def _b(s):
    return jnp.broadcast_to(s, (16,))


def _take(v, idx):
    return v.at[idx].get(mode=_INB)


def _ln(x):
    bits = lax.bitcast_convert_type(x, _i32)
    e = (bits >> 23) & 0xFF
    m = lax.bitcast_convert_type((bits & 0x007FFFFF) | 0x3F800000, _f32)
    big = m >= _f32(SQRT2)
    m = jnp.where(big, m * _f32(0.5), m)
    ef = (e - 127).astype(_f32) + jnp.where(big, _f32(1.0), _f32(0.0))
    t = (m - _f32(1.0)) / (m + _f32(1.0))
    t2 = t * t
    p = t * (_f32(2.0) + t2 * (_f32(2.0 / 3.0) + t2 * (_f32(2.0 / 5.0)
             + t2 * (_f32(2.0 / 7.0) + t2 * _f32(2.0 / 9.0)))))
    return p + ef * _f32(LN2)


def _sl1(d):
    a = jnp.abs(d)
    m = jnp.minimum(a, _f32(1.0))
    return _f32(0.5) * m * (a + a - m)


def _body(stkh, gflath,
          loss_out, idx_out,
          rxv, ryv, rx2v, ry2v, axv, ayv, ax2v, ay2v, scv,
          g4v, ioub, rmaxb, partv, shared, allv,
          lossov, idxov, sem):
    cid = lax.axis_index("c")
    sid = lax.axis_index("s")

    @pl.when(cid == 0)
    def _core0():
        li = lax.iota(_i32, 16)
        lif = li.astype(_f32)

        base = sid * PER_TILE
        basef = base.astype(_f32)
        cps = [
            pltpu.async_copy(stkh.at[pl.ds(0 * NP + base, PER_TILE)], rxv, sem),
            pltpu.async_copy(stkh.at[pl.ds(1 * NP + base, PER_TILE)], ryv, sem),
            pltpu.async_copy(stkh.at[pl.ds(2 * NP + base, PER_TILE)], rx2v, sem),
            pltpu.async_copy(stkh.at[pl.ds(3 * NP + base, PER_TILE)], ry2v, sem),
            pltpu.async_copy(stkh.at[pl.ds(4 * NP + base, PER_TILE)], axv, sem),
            pltpu.async_copy(stkh.at[pl.ds(5 * NP + base, PER_TILE)], ayv, sem),
            pltpu.async_copy(stkh.at[pl.ds(6 * NP + base, PER_TILE)], ax2v, sem),
            pltpu.async_copy(stkh.at[pl.ds(7 * NP + base, PER_TILE)], ay2v, sem),
            pltpu.async_copy(stkh.at[pl.ds(8 * NP + base, PER_TILE)], scv, sem),
            pltpu.async_copy(gflath, g4v.at[pl.ds(0, 64)], sem),
        ]
        for c in cps:
            c.wait()

        i4 = li * 4
        gx1 = plsc.load_gather(g4v, [i4])
        gy1 = plsc.load_gather(g4v, [i4 + 1])
        gx2 = plsc.load_gather(g4v, [i4 + 2])
        gy2 = plsc.load_gather(g4v, [i4 + 3])
        gw = gx2 - gx1
        gh = gy2 - gy1
        gav = gw * gh
        gcx = (gx1 + gx2) * _f32(0.5)
        gcy = (gy1 + gy2) * _f32(0.5)
        lgw = _ln(gw)
        lgh = _ln(gh)

        ones = _b(_f32(1.0))
        zeros = _b(_f32(0.0))
        cgs = [jnp.full((16,), g, _i32) for g in range(G)]


        def slow(cnt_a, regsum_a, cx1, cy1, cx2, cy2,
                 bx1, by1, bx2, by2, valid, start):
            rw = cx2 - cx1
            rh = cy2 - cy1
            rcx = (cx1 + cx2) * _f32(0.5)
            rcy = (cy1 + cy2) * _f32(0.5)
            lrw = _ln(jnp.where(valid, rw, ones))
            lrh = _ln(jnp.where(valid, rh, ones))
            invaw = ones / jnp.where(valid, bx2 - bx1, ones)
            invah = ones / jnp.where(valid, by2 - by1, ones)
            for g in range(G):
                iou = ioub[pl.ds(g * PER_TILE + start, 16)]
                pos = iou > _f32(0.8)
                cnt_a = cnt_a + jnp.where(pos, ones, zeros)
                dx = (rcx - _take(gcx, cgs[g])) * invaw
                dy = (rcy - _take(gcy, cgs[g])) * invah
                dw = lrw - _take(lgw, cgs[g])
                dh = lrh - _take(lgh, cgs[g])
                pp = (_sl1(dx) + _sl1(dy) + _sl1(dw) + _sl1(dh)) * _f32(0.25)
                regsum_a = regsum_a + jnp.where(pos, pp, zeros)
            return cnt_a, regsum_a

        def fast(cnt_a, regsum_a, cx1, cy1, cx2, cy2,
                 bx1, by1, bx2, by2, valid, start):
            return cnt_a, regsum_a

        def chunk(c, carry):
            (cnt_a, regsum_a, npos_a, nneg_a, clspos_a) = carry
            start = pl.multiple_of(c * 16, 16)
            rowf = lif + basef + start.astype(_f32)
            valid = rowf < _f32(N)

            cx1 = rxv[pl.ds(start, 16)]
            cy1 = ryv[pl.ds(start, 16)]
            cx2 = rx2v[pl.ds(start, 16)]
            cy2 = ry2v[pl.ds(start, 16)]
            bx1 = axv[pl.ds(start, 16)]
            by1 = ayv[pl.ds(start, 16)]
            bx2 = ax2v[pl.ds(start, 16)]
            by2 = ay2v[pl.ds(start, 16)]
            scc = scv[pl.ds(start, 16)]
            arear = (cx2 - cx1) * (cy2 - cy1)

            rmax = -ones
            for g in range(G):
                iw = (jnp.minimum(cx2, _take(gx2, cgs[g]))
                      - jnp.maximum(cx1, _take(gx1, cgs[g])))
                ih = (jnp.minimum(cy2, _take(gy2, cgs[g]))
                      - jnp.maximum(cy1, _take(gy1, cgs[g])))
                inter = jnp.maximum(iw, zeros) * jnp.maximum(ih, zeros)
                iou = inter / (arear + _take(gav, cgs[g]) - inter)
                ioub[pl.ds(g * PER_TILE + start, 16)] = iou
                rmax = jnp.maximum(rmax, iou)
            rmaxb[pl.ds(start, 16)] = rmax

            pv = (rmax > _f32(0.8)) & valid
            nv = (rmax < _f32(0.3)) & valid
            npos_a = npos_a + jnp.where(pv, ones, zeros)
            clspos_a = clspos_a + jnp.where(pv, _f32(C1) - scc, zeros)
            nneg_a = nneg_a + jnp.where(nv, ones, zeros)

            cnt_a, regsum_a = lax.cond(
                jnp.any(rmax > _f32(0.8)), slow, fast,
                cnt_a, regsum_a, cx1, cy1, cx2, cy2,
                bx1, by1, bx2, by2, valid, start)
            return (cnt_a, regsum_a, npos_a, nneg_a, clspos_a)

        init = (zeros, zeros, zeros, zeros, zeros)
        (cnt_a, regsum_a, npos_a, nneg_a, clspos_a) = \
            lax.fori_loop(0, CHUNKS, chunk, init)

        colmax = -ones
        colloc = zeros
        big = _b(_f32(1e9))
        for g in range(G):
            maxv = ioub[pl.ds(g * PER_TILE, 16)]
            idxv = lif
            for c in range(1, CHUNKS):
                v = ioub[pl.ds(g * PER_TILE + c * 16, 16)]
                u = v > maxv
                maxv = jnp.where(u, v, maxv)
                idxv = jnp.where(u, lif + _f32(c * 16), idxv)
            m = jnp.max(maxv)
            bm = _b(m)
            fi = jnp.min(jnp.where(maxv >= bm, idxv, big))
            lm = li == g
            colmax = jnp.where(lm, bm, colmax)
            colloc = jnp.where(lm, _b(fi), colloc)

        lrowi = colloc.astype(_i32)
        colidx = colloc + _b(basef)
        colrm = plsc.load_gather(rmaxb, [lrowi])
        colsc = plsc.load_gather(scv, [lrowi])

        px1 = plsc.load_gather(rxv, [lrowi])
        py1 = plsc.load_gather(ryv, [lrowi])
        px2 = plsc.load_gather(rx2v, [lrowi])
        py2 = plsc.load_gather(ry2v, [lrowi])
        qx1 = plsc.load_gather(axv, [lrowi])
        qy1 = plsc.load_gather(ayv, [lrowi])
        qx2 = plsc.load_gather(ax2v, [lrowi])
        qy2 = plsc.load_gather(ay2v, [lrowi])
        aw = qx2 - qx1
        ah = qy2 - qy1
        dx = ((px1 + px2) * _f32(0.5) - gcx) / aw
        dy = ((py1 + py2) * _f32(0.5) - gcy) / ah
        dw = _ln(px2 - px1) - lgw
        dh = _ln(py2 - py1) - lgh
        colpp = (_sl1(dx) + _sl1(dy) + _sl1(dw) + _sl1(dh)) * _f32(0.25)

        partv[pl.ds(0, 16)] = colmax
        partv[pl.ds(16, 16)] = colidx
        partv[pl.ds(32, 16)] = colpp
        partv[pl.ds(48, 16)] = colsc
        partv[pl.ds(64, 16)] = colrm
        partv[pl.ds(80, 16)] = cnt_a
        partv[pl.ds(96, 16)] = regsum_a
        partv[pl.ds(112, 16)] = npos_a
        partv[pl.ds(128, 16)] = nneg_a
        partv[pl.ds(144, 16)] = clspos_a
        pltpu.sync_copy(partv, shared.at[pl.ds(sid * 160, 160)])
        plsc.subcore_barrier()

        @pl.when(sid == 0)
        def _final():
            pltpu.sync_copy(shared, allv)
            mcolmax = allv[pl.ds(0, 16)]
            mcolidx = allv[pl.ds(16, 16)]
            mcolpp = allv[pl.ds(32, 16)]
            mcolsc = allv[pl.ds(48, 16)]
            mcolrm = allv[pl.ds(64, 16)]
            mcnt = allv[pl.ds(80, 16)]
            mregsum = allv[pl.ds(96, 16)]
            mnpos = allv[pl.ds(112, 16)]
            mnneg = allv[pl.ds(128, 16)]
            mclspos = allv[pl.ds(144, 16)]
            for t in range(1, 16):
                tm = allv[pl.ds(t * 160 + 0, 16)]
                u = tm > mcolmax
                mcolmax = jnp.where(u, tm, mcolmax)
                mcolidx = jnp.where(u, allv[pl.ds(t * 160 + 16, 16)], mcolidx)
                mcolpp = jnp.where(u, allv[pl.ds(t * 160 + 32, 16)], mcolpp)
                mcolsc = jnp.where(u, allv[pl.ds(t * 160 + 48, 16)], mcolsc)
                mcolrm = jnp.where(u, allv[pl.ds(t * 160 + 64, 16)], mcolrm)
                mcnt = mcnt + allv[pl.ds(t * 160 + 80, 16)]
                mregsum = mregsum + allv[pl.ds(t * 160 + 96, 16)]
                mnpos = mnpos + allv[pl.ds(t * 160 + 112, 16)]
                mnneg = mnneg + allv[pl.ds(t * 160 + 128, 16)]
                mclspos = mclspos + allv[pl.ds(t * 160 + 144, 16)]

            best = mcolidx.astype(_i32)

            notpos = mcolmax <= _f32(0.8)
            d_cntr = jnp.sum(jnp.where(notpos, ones, zeros))
            d_regsum = jnp.sum(jnp.where(notpos, mcolpp, zeros))

            dup = li < 0
            for k in range(G - 1):
                bk = _b(best[k])
                dup = dup | ((best == bk) & (li > k))
            firstm = ~dup

            anyposb = mcolrm > _f32(0.8)
            allnegb = mcolrm < _f32(0.3)
            notany = firstm & (~anyposb)
            d_npos = jnp.sum(jnp.where(notany, ones, zeros))
            d_clspos = jnp.sum(jnp.where(notany, _f32(C1) - mcolsc, zeros))
            d_nneg = jnp.sum(jnp.where(firstm & allnegb, ones, zeros))

            npos_v = _b(jnp.sum(mnpos)) + _b(d_npos)
            nneg_v = _b(jnp.sum(mnneg)) - _b(d_nneg)
            nsel = npos_v + nneg_v
            cls = _b(jnp.sum(mclspos)) + _b(d_clspos) + nneg_v * _f32(C2)
            cntr = _b(jnp.sum(mcnt)) + _b(d_cntr)
            regs = _b(jnp.sum(mregsum)) + _b(d_regsum)
            lossv = cls / nsel / nsel + _f32(10.0) * regs / cntr

            lossov[...] = lossv
            idxov[...] = _b(best[G - 1])
            pltpu.sync_copy(lossov, loss_out)
            pltpu.sync_copy(idxov, idx_out)


@jax.jit
def kernel(scores, regions, anchors, ground_truth_boxes):
    cols = jnp.stack([regions[:, 0], regions[:, 1],
                      regions[:, 2], regions[:, 3],
                      anchors[:, 0], anchors[:, 1],
                      anchors[:, 2], anchors[:, 3], scores])
    stk = jnp.zeros((9, NP), _f32).at[:, :N].set(cols).reshape(-1)
    gflat = ground_truth_boxes.reshape(-1)

    mesh = plsc.VectorSubcoreMesh(core_axis_name="c", subcore_axis_name="s",
                                  num_cores=1)
    f = pl.kernel(
        _body,
        out_type=(
            jax.ShapeDtypeStruct((16,), _f32),
            jax.ShapeDtypeStruct((16,), _i32),
        ),
        mesh=mesh,
        compiler_params=pltpu.CompilerParams(needs_layout_passes=False,
                                             skip_device_barrier=True),
        scratch_types=[
            pltpu.VMEM((PER_TILE,), _f32),
            pltpu.VMEM((PER_TILE,), _f32),
            pltpu.VMEM((PER_TILE,), _f32),
            pltpu.VMEM((PER_TILE,), _f32),
            pltpu.VMEM((PER_TILE,), _f32),
            pltpu.VMEM((PER_TILE,), _f32),
            pltpu.VMEM((PER_TILE,), _f32),
            pltpu.VMEM((PER_TILE,), _f32),
            pltpu.VMEM((PER_TILE,), _f32),
            pltpu.VMEM((64,), _f32),
            pltpu.VMEM((G * PER_TILE,), _f32),
            pltpu.VMEM((PER_TILE,), _f32),
            pltpu.VMEM((160,), _f32),
            pltpu.VMEM_SHARED((2560,), _f32),
            pltpu.VMEM((2560,), _f32),
            pltpu.VMEM((16,), _f32),
            pltpu.VMEM((16,), _i32),
            pltpu.SemaphoreType.DMA,
        ],
    )
    loss_v, idx_v = f(stk, gflat)
    return loss_v[0], idx_v[0]
